# baseline (speedup 1.0000x reference)
import functools

import jax
import jax.numpy as jnp
from jax import lax
from jax.experimental import pallas as pl
from jax.experimental.pallas import tpu as pltpu
from jax.experimental.pallas import tpu_sc as plsc

MU = 3.5
BU = 0.1
BI = -0.05
B_UI = MU + BU + BI

N_USER = 10000
N_ITEM = 1000
H = 64
W = 128
BATCH = 16384

NUM_CORES = 2
NUM_SUBCORES = 16
L = 16
NW = NUM_CORES * NUM_SUBCORES
BPW = BATCH // NW
CHUNK = 256

DEPAD_ROWS = 200
NBLK = (N_USER + W - 1) // W


def _depad_body(w_ref, o_ref):
    x = w_ref[...]
    parts = [x[:, b * W:(b + 1) * W] for b in range(N_USER // W)]
    parts.append(jnp.pad(x[:, (N_USER // W) * W:],
                         ((0, 0), (0, NBLK * W - N_USER))))
    o_ref[...] = jnp.stack(parts, axis=0)


def _mf_body(u_hbm, i_hbm, p_hbm, q_hbm, mf_hbm,
             u_v, i_v, p_v, q_v, o_v, sem):
    wid = lax.axis_index("s") * NUM_CORES + lax.axis_index("c")
    base = wid * BPW

    pltpu.sync_copy(u_hbm.at[pl.ds(base, BPW)], u_v)
    pltpu.sync_copy(i_hbm.at[pl.ds(base, BPW)], i_v)

    last_lane = lax.iota(jnp.int32, L) == (L - 1)

    def chunk_body(c, carry):
        cbase = c * CHUNK
        cp_p = pltpu.async_copy(p_hbm.at[u_v.at[pl.ds(cbase, CHUNK)]], p_v,
                                sem)
        cp_q = pltpu.async_copy(q_hbm.at[i_v.at[pl.ds(cbase, CHUNK)]], q_v,
                                sem)
        cp_p.wait()
        cp_q.wait()

        def row_body(b, carry2):
            acc = p_v[b, pl.ds(0, L)] * q_v[b, pl.ds(0, L)]
            for h in range(1, H // L):
                acc = acc + p_v[b, pl.ds(h * L, L)] * q_v[b, pl.ds(h * L, L)]
            tot = plsc.cumsum(acc)
            plsc.store_scatter(o_v, [jnp.full((L,), cbase + b, jnp.int32)],
                               tot, mask=last_lane)
            return carry2

        lax.fori_loop(0, CHUNK, row_body, 0)
        return carry

    lax.fori_loop(0, BPW // CHUNK, chunk_body, 0)
    pltpu.sync_copy(o_v, mf_hbm.at[pl.ds(base, BPW)])


def _w_body(u_hbm, i_hbm, r_hbm, mf_hbm, wdp_hbm, out_hbm,
            u_v, i_v, r_v, mf_v, row_v, o_v, wbuf_v, sem):
    wid = lax.axis_index("s") * NUM_CORES + lax.axis_index("c")
    base = wid * BPW

    pltpu.sync_copy(u_hbm.at[pl.ds(base, BPW)], u_v)
    pltpu.sync_copy(i_hbm.at[pl.ds(base, BPW)], i_v)
    pltpu.sync_copy(r_hbm.at[pl.ds(base, BPW)], r_v)
    pltpu.sync_copy(mf_hbm.at[pl.ds(base, BPW)], mf_v)

    def rows_body(g, carry):
        s = pl.ds(g * L, L)
        row_v[s] = (jax.lax.shift_right_logical(u_v[s], 7) * N_ITEM
                    + i_v[s])
        return carry

    lax.fori_loop(0, BPW // L, rows_body, 0)

    lanes = lax.iota(jnp.int32, L)

    def chunk_body(c, carry):
        cbase = c * CHUNK
        cp = pltpu.async_copy(wdp_hbm.at[row_v.at[pl.ds(cbase, CHUNK)]],
                              wbuf_v, sem)
        cp.wait()

        def sel_body(g, carry2):
            s = pl.ds(cbase + g * L, L)
            colv = u_v[s] & (W - 1)
            val = plsc.load_gather(wbuf_v, [g * L + lanes, colv])
            o_v[s] = mf_v[s] + B_UI + val * (r_v[s] - B_UI)
            return carry2

        lax.fori_loop(0, CHUNK // L, sel_body, 0)
        return carry

    lax.fori_loop(0, BPW // CHUNK, chunk_body, 0)
    pltpu.sync_copy(o_v, out_hbm.at[pl.ds(base, BPW)])


@jax.jit
def _svd_sc(u, i, r, w_ij, P, Q):
    p2 = jnp.concatenate([P, P], axis=1)
    q2 = jnp.concatenate([Q, Q], axis=1)
    wdp3 = pl.pallas_call(
        _depad_body,
        grid=(N_ITEM // DEPAD_ROWS,),
        in_specs=[pl.BlockSpec((DEPAD_ROWS, N_USER), lambda r: (r, 0))],
        out_specs=pl.BlockSpec((NBLK, DEPAD_ROWS, W), lambda r: (0, r, 0)),
        out_shape=jax.ShapeDtypeStruct((NBLK, N_ITEM, W), jnp.float32),
    )(w_ij.T)
    wdp = wdp3.reshape(NBLK * N_ITEM, W)

    mesh = plsc.VectorSubcoreMesh(core_axis_name="c", subcore_axis_name="s")
    sc_params = pltpu.CompilerParams(needs_layout_passes=False)

    mf = functools.partial(
        pl.kernel,
        mesh=mesh,
        compiler_params=sc_params,
        out_type=jax.ShapeDtypeStruct((BATCH,), jnp.float32),
        scratch_types=[
            pltpu.VMEM((BPW,), jnp.int32),
            pltpu.VMEM((BPW,), jnp.int32),
            pltpu.VMEM((CHUNK, W), jnp.float32),
            pltpu.VMEM((CHUNK, W), jnp.float32),
            pltpu.VMEM((BPW,), jnp.float32),
            pltpu.SemaphoreType.DMA,
        ],
    )(_mf_body)(u, i, p2, q2)

    out = functools.partial(
        pl.kernel,
        mesh=mesh,
        compiler_params=sc_params,
        out_type=jax.ShapeDtypeStruct((BATCH,), jnp.float32),
        scratch_types=[
            pltpu.VMEM((BPW,), jnp.int32),
            pltpu.VMEM((BPW,), jnp.int32),
            pltpu.VMEM((BPW,), jnp.float32),
            pltpu.VMEM((BPW,), jnp.float32),
            pltpu.VMEM((BPW,), jnp.int32),
            pltpu.VMEM((BPW,), jnp.float32),
            pltpu.VMEM((CHUNK, W), jnp.float32),
            pltpu.SemaphoreType.DMA,
        ],
    )(_w_body)(u, i, r, mf, wdp)

    return out


def kernel(u, i, r, w_ij, P, Q):
    u = u.astype(jnp.int32)
    i = i.astype(jnp.int32)
    return _svd_sc(u, i, r, w_ij, P, Q)


# speedup vs baseline: 1.0117x; 1.0117x over previous
import functools

import jax
import jax.numpy as jnp
from jax import lax
from jax.experimental import pallas as pl
from jax.experimental.pallas import tpu as pltpu
from jax.experimental.pallas import tpu_sc as plsc

MU = 3.5
BU = 0.1
BI = -0.05
B_UI = MU + BU + BI

N_USER = 10000
N_ITEM = 1000
H = 64
W = 128
BATCH = 16384

NUM_CORES = 2
NUM_SUBCORES = 16
L = 16
NW = NUM_CORES * NUM_SUBCORES
BPW = BATCH // NW
CHUNK = 256

DEPAD_ROWS = 40
NBLK = (N_USER + W - 1) // W


def _depad_body(w_ref, o_ref):
    x = w_ref[...]
    parts = [x[:, b * W:(b + 1) * W] for b in range(N_USER // W)]
    parts.append(jnp.pad(x[:, (N_USER // W) * W:],
                         ((0, 0), (0, NBLK * W - N_USER))))
    o_ref[...] = jnp.stack(parts, axis=0)


def _mf_body(u_hbm, i_hbm, p_hbm, q_hbm, mf_hbm,
             u_v, i_v, p_v, q_v, o_v, sem):
    wid = lax.axis_index("s") * NUM_CORES + lax.axis_index("c")
    base = wid * BPW

    pltpu.sync_copy(u_hbm.at[pl.ds(base, BPW)], u_v)
    pltpu.sync_copy(i_hbm.at[pl.ds(base, BPW)], i_v)

    last_lane = lax.iota(jnp.int32, L) == (L - 1)

    def chunk_body(c, carry):
        cbase = c * CHUNK
        cp_p = pltpu.async_copy(p_hbm.at[u_v.at[pl.ds(cbase, CHUNK)]], p_v,
                                sem)
        cp_q = pltpu.async_copy(q_hbm.at[i_v.at[pl.ds(cbase, CHUNK)]], q_v,
                                sem)
        cp_p.wait()
        cp_q.wait()

        def row_body(b, carry2):
            acc = p_v[b, pl.ds(0, L)] * q_v[b, pl.ds(0, L)]
            for h in range(1, H // L):
                acc = acc + p_v[b, pl.ds(h * L, L)] * q_v[b, pl.ds(h * L, L)]
            tot = plsc.cumsum(acc)
            plsc.store_scatter(o_v, [jnp.full((L,), cbase + b, jnp.int32)],
                               tot, mask=last_lane)
            return carry2

        lax.fori_loop(0, CHUNK, row_body, 0)
        return carry

    lax.fori_loop(0, BPW // CHUNK, chunk_body, 0)
    pltpu.sync_copy(o_v, mf_hbm.at[pl.ds(base, BPW)])


def _w_body(u_hbm, i_hbm, r_hbm, mf_hbm, wdp_hbm, out_hbm,
            u_v, i_v, r_v, mf_v, row_v, o_v, wbuf_v, sem):
    wid = lax.axis_index("s") * NUM_CORES + lax.axis_index("c")
    base = wid * BPW

    pltpu.sync_copy(u_hbm.at[pl.ds(base, BPW)], u_v)
    pltpu.sync_copy(i_hbm.at[pl.ds(base, BPW)], i_v)
    pltpu.sync_copy(r_hbm.at[pl.ds(base, BPW)], r_v)
    pltpu.sync_copy(mf_hbm.at[pl.ds(base, BPW)], mf_v)

    def rows_body(g, carry):
        s = pl.ds(g * L, L)
        row_v[s] = (jax.lax.shift_right_logical(u_v[s], 7) * N_ITEM
                    + i_v[s])
        return carry

    lax.fori_loop(0, BPW // L, rows_body, 0)

    lanes = lax.iota(jnp.int32, L)

    def chunk_body(c, carry):
        cbase = c * CHUNK
        cp = pltpu.async_copy(wdp_hbm.at[row_v.at[pl.ds(cbase, CHUNK)]],
                              wbuf_v, sem)
        cp.wait()

        def sel_body(g, carry2):
            s = pl.ds(cbase + g * L, L)
            colv = u_v[s] & (W - 1)
            val = plsc.load_gather(wbuf_v, [g * L + lanes, colv])
            o_v[s] = mf_v[s] + B_UI + val * (r_v[s] - B_UI)
            return carry2

        lax.fori_loop(0, CHUNK // L, sel_body, 0)
        return carry

    lax.fori_loop(0, BPW // CHUNK, chunk_body, 0)
    pltpu.sync_copy(o_v, out_hbm.at[pl.ds(base, BPW)])


@jax.jit
def _svd_sc(u, i, r, w_ij, P, Q):
    p2 = jnp.concatenate([P, P], axis=1)
    q2 = jnp.concatenate([Q, Q], axis=1)
    wdp3 = pl.pallas_call(
        _depad_body,
        grid=(N_ITEM // DEPAD_ROWS,),
        in_specs=[pl.BlockSpec((DEPAD_ROWS, N_USER), lambda r: (r, 0))],
        out_specs=pl.BlockSpec((NBLK, DEPAD_ROWS, W), lambda r: (0, r, 0)),
        out_shape=jax.ShapeDtypeStruct((NBLK, N_ITEM, W), jnp.float32),
    )(w_ij.T)
    wdp = wdp3.reshape(NBLK * N_ITEM, W)

    mesh = plsc.VectorSubcoreMesh(core_axis_name="c", subcore_axis_name="s")
    sc_params = pltpu.CompilerParams(needs_layout_passes=False)

    mf = functools.partial(
        pl.kernel,
        mesh=mesh,
        compiler_params=sc_params,
        out_type=jax.ShapeDtypeStruct((BATCH,), jnp.float32),
        scratch_types=[
            pltpu.VMEM((BPW,), jnp.int32),
            pltpu.VMEM((BPW,), jnp.int32),
            pltpu.VMEM((CHUNK, W), jnp.float32),
            pltpu.VMEM((CHUNK, W), jnp.float32),
            pltpu.VMEM((BPW,), jnp.float32),
            pltpu.SemaphoreType.DMA,
        ],
    )(_mf_body)(u, i, p2, q2)

    out = functools.partial(
        pl.kernel,
        mesh=mesh,
        compiler_params=sc_params,
        out_type=jax.ShapeDtypeStruct((BATCH,), jnp.float32),
        scratch_types=[
            pltpu.VMEM((BPW,), jnp.int32),
            pltpu.VMEM((BPW,), jnp.int32),
            pltpu.VMEM((BPW,), jnp.float32),
            pltpu.VMEM((BPW,), jnp.float32),
            pltpu.VMEM((BPW,), jnp.int32),
            pltpu.VMEM((BPW,), jnp.float32),
            pltpu.VMEM((CHUNK, W), jnp.float32),
            pltpu.SemaphoreType.DMA,
        ],
    )(_w_body)(u, i, r, mf, wdp)

    return out


def kernel(u, i, r, w_ij, P, Q):
    u = u.astype(jnp.int32)
    i = i.astype(jnp.int32)
    return _svd_sc(u, i, r, w_ij, P, Q)
